# Initial kernel scaffold; baseline (speedup 1.0000x reference)
import jax
import jax.numpy as jnp
from jax.experimental import pallas as pl


def kernel(x, w_minute, w_hour, w_weekday, w_day, w_month):
    raise NotImplementedError("write your pallas kernel here")


# speedup vs baseline: 19.1370x; 19.1370x over previous
import jax
import jax.numpy as jnp
from jax.experimental import pallas as pl
from jax.experimental.pallas import tpu as pltpu

_P = 1024
_K = 64


def _body(ctr_ref, w_ref, out_ref):
    c = ctr_ref[...]
    iota = jax.lax.broadcasted_iota(jnp.int32, (_P, _K), 1)
    acc = jnp.zeros((_P, _K), jnp.float32)
    for j in range(8):
        acc += (c[j, :, None] == iota).astype(jnp.float32)
    out_ref[...] = jnp.dot(acc, w_ref[...], preferred_element_type=jnp.float32)


def kernel(x, w_minute, w_hour, w_weekday, w_day, w_month):
    d_model = w_minute.shape[1]
    n = x.shape[0] * x.shape[1]

    def first8(w):
        r = w[:8]
        if r.shape[0] < 8:
            r = jnp.pad(r, ((0, 8 - r.shape[0]), (0, 0)))
        return r

    w = jnp.concatenate(
        [first8(w_month), first8(w_day), first8(w_weekday), first8(w_hour),
         first8(w_minute), first8(w_minute),
         jnp.zeros((_K - 48, d_model), jnp.float32)], axis=0)

    c = x.reshape(n, 6).astype(jnp.int32) + jnp.arange(6, dtype=jnp.int32) * 8
    ctr = jnp.concatenate(
        [c.T, jnp.full((2, n), 48, jnp.int32)], axis=0)

    out = pl.pallas_call(
        _body,
        grid=(n // _P,),
        in_specs=[
            pl.BlockSpec((8, _P), lambda i: (0, i)),
            pl.BlockSpec((_K, d_model), lambda i: (0, 0)),
        ],
        out_specs=pl.BlockSpec((_P, d_model), lambda i: (i, 0)),
        out_shape=jax.ShapeDtypeStruct((n, d_model), jnp.float32),
        compiler_params=pltpu.CompilerParams(
            dimension_semantics=("arbitrary",)),
    )(ctr, w)
    return out.reshape(x.shape[0], x.shape[1], d_model)
